# baseline (speedup 1.0000x reference)
import functools

import jax
import jax.numpy as jnp
from jax import lax
from jax.experimental import pallas as pl
from jax.experimental.pallas import tpu as pltpu
from jax.experimental.pallas import tpu_sc as plsc


@functools.lru_cache(maxsize=None)
def _make_sc_aggr(N, E, H):
    info = plsc.get_sparse_core_info()
    NC, NS, L = info.num_cores, info.num_subcores, info.num_lanes
    NW = NC * NS
    PER_W = E // NW
    C = 40
    NCH = PER_W // C
    assert PER_W % C == 0 and E % NW == 0
    NB = 3
    MB = 2
    D = 6
    UN = 6
    J = (NCH - 4) // UN
    J = NCH // UN
    TAIL = NCH - J * UN
    BLK = 40
    NBLK = N // BLK
    KMAX = (NBLK + NS - 1) // NS
    assert N % BLK == 0
    QS = H // L

    mesh = plsc.VectorSubcoreMesh(core_axis_name="c", subcore_axis_name="s")

    @functools.partial(
        pl.kernel,
        out_type=jax.ShapeDtypeStruct((NC, N, H), jnp.float32),
        mesh=mesh,
        scratch_types=[
            pltpu.VMEM((NB, C), jnp.int32),
            pltpu.VMEM((D, C), jnp.int32),
            pltpu.VMEM((NB, C, H), jnp.float32),
            pltpu.VMEM((NB, C, H), jnp.float32),
            pltpu.VMEM((MB, C, H), jnp.float32),
            pltpu.VMEM_SHARED((N, H), jnp.float32),
        ] + [pltpu.SemaphoreType.DMA] * 17,
    )
    def sc_aggr(h_hbm, e_hbm, src_hbm, dst_hbm, out_hbm,
                sring, dring, hring, ering, mring, accum, *sems):
        semsi = sems[0:3]
        semse = sems[3:6]
        semdi = sems[6:12]
        semg = sems[12:15]
        semsc = sems[15:17]
        cid = lax.axis_index("c")
        sid = lax.axis_index("s")
        wid = sid * NC + cid
        base = wid * PER_W

        zv = jnp.zeros((L,), jnp.float32)
        stage = mring.at[1]

        def zero_stage(r, carry):
            for q in range(QS):
                stage[r, pl.ds(q * L, L)] = zv
            return carry

        lax.fori_loop(0, BLK, zero_stage, 0)

        def zero_accum(k, carry):
            blk = sid + k * NS

            @pl.when(blk < NBLK)
            def _():
                pltpu.sync_copy(stage, accum.at[pl.ds(blk * BLK, BLK)])
            return carry

        def sidx_desc(c):
            b = c % NB if isinstance(c, int) else None
            return pltpu.make_async_copy(
                src_hbm.at[pl.ds(base + c * C, C)], sring.at[c % NB],
                semsi[c % NB])

        def didx_desc(c):
            return pltpu.make_async_copy(
                dst_hbm.at[pl.ds(base + c * C, C)], dring.at[c % D],
                semdi[c % D])

        def eload_desc(c):
            return pltpu.make_async_copy(
                e_hbm.at[pl.ds(base + c * C, C)], ering.at[c % NB],
                semse[c % NB])

        def gather_desc(c):
            return pltpu.make_async_copy(
                h_hbm.at[sring.at[c % NB]], hring.at[c % NB], semg[c % NB])

        def scat_desc(c):
            return pltpu.make_async_copy(
                mring.at[c % MB], accum.at[dring.at[c % D]], semsc[c % MB])

        def issue_loads(c):
            sidx_desc(c).start()
            didx_desc(c).start()
            eload_desc(c).start()

        def phase(c, first=False):
            if c2ok(c):
                sidx_desc(c + 2).wait()
                gather_desc(c + 2).start()
            gather_desc(c).wait()
            eload_desc(c).wait()
            if first:
                @pl.when(cval(c) >= MB)
                def _():
                    scat_desc(c - MB).wait()
            else:
                scat_desc(c - MB).wait()
            hb = hring.at[c % NB]
            eb = ering.at[c % NB]
            mb = mring.at[c % MB]
            def rowfn(r, rc):
                for q in range(QS):
                    sl = pl.ds(q * L, L)
                    mb[r, sl] = jnp.maximum(hb[r, sl] + eb[r, sl], 0.0)
                return rc

            lax.fori_loop(0, C, rowfn, 0)
            didx_desc(c).wait()
            scat_desc(c).start(add=True)
            if c3ok(c):
                issue_loads(c + 3)

        for c in range(NB):
            issue_loads(c)
        lax.fori_loop(0, KMAX, zero_accum, 0)
        plsc.subcore_barrier()
        for c in range(2):
            sidx_desc(c).wait()
            gather_desc(c).start()

        def loop_body(j, carry):
            c0 = j * UN
            for k in range(UN):
                phase(_Shifted(c0, k), first=(k < MB))
            return carry

        lax.fori_loop(0, J, loop_body, 0)
        for k in range(TAIL):
            phase(J * UN + k)

        for k in range(MB):
            scat_desc(NCH - MB + k).wait()
        plsc.subcore_barrier()

        def drain(k, carry):
            blk = sid + k * NS

            @pl.when(blk < NBLK)
            def _():
                r0 = blk * BLK
                pltpu.sync_copy(accum.at[pl.ds(r0, BLK)], stage)
                pltpu.sync_copy(stage, out_hbm.at[cid, pl.ds(r0, BLK)])
            return carry

        lax.fori_loop(0, KMAX, drain, 0)

    class _Shifted:
        def __init__(self, c0, k):
            self.c0 = c0
            self.k = k

        def __mod__(self, m):
            assert UN % m == 0
            return self.k % m

        def __add__(self, o):
            return _Shifted(self.c0, self.k + o)

        def __sub__(self, o):
            return _Shifted(self.c0, self.k - o)

        def __mul__(self, o):
            return (self.c0 + self.k) * o

        def __rmul__(self, o):
            return self.__mul__(o)

    def cval(c):
        return c.c0 + c.k if isinstance(c, _Shifted) else c

    def c2ok(c):
        if isinstance(c, _Shifted):
            return True
        return c + 2 < NCH

    def c3ok(c):
        if isinstance(c, _Shifted):
            return True
        return c + 3 < NCH

    return sc_aggr


def _edge_mlp(edge_attr, We, be):
    E, DE = edge_attr.shape
    H = We.shape[1]
    BE = 2000
    grid = (E // BE,)

    def body(ea, w, b, o):
        o[...] = jnp.dot(ea[...], w[...],
                         preferred_element_type=jnp.float32) + b[...]

    return pl.pallas_call(
        body,
        grid=grid,
        in_specs=[pl.BlockSpec((BE, DE), lambda i: (i, 0)),
                  pl.BlockSpec((DE, H), lambda i: (0, 0)),
                  pl.BlockSpec((1, H), lambda i: (0, 0))],
        out_specs=pl.BlockSpec((BE, H), lambda i: (i, 0)),
        out_shape=jax.ShapeDtypeStruct((E, H), jnp.float32),
    )(edge_attr, We, be.reshape(1, H))


def _node_fused(h, aggr, batch2, G, W1, b1, W2, b2, gw, gb, gms, Wl=None,
                bl=None):
    N, H = h.shape
    BN = 2000
    NBK = N // BN
    final = Wl is not None

    def body(*refs):
        if final:
            (h_ref, a_ref, bt_ref, w1, b1_, w2, b2_, gw_, gb_, gms_,
             wl, bl_, out, t_ref) = refs
        else:
            (h_ref, a_ref, bt_ref, w1, b1_, w2, b2_, gw_, gb_, gms_,
             out, t_ref) = refs

        def onehot_blk(i):
            bt = bt_ref[pl.ds(i * BN, BN), :]
            ot = lax.broadcasted_iota(jnp.int32, (BN, G), 1)
            return (ot == bt).astype(jnp.float32)

        def blk1(i, carry):
            s1, s2, cnt = carry
            sl = pl.ds(i * BN, BN)
            z = h_ref[sl, :] + a_ref[0, sl, :] + a_ref[1, sl, :]
            t = jnp.dot(
                jnp.maximum(jnp.dot(z, w1[...],
                                    preferred_element_type=jnp.float32)
                            + b1_[...], 0.0),
                w2[...], preferred_element_type=jnp.float32) + b2_[...]
            t_ref[sl, :] = t
            OT = onehot_blk(i)
            dn = (((0,), (0,)), ((), ()))
            s1 = s1 + lax.dot_general(OT, t, dn,
                                      preferred_element_type=jnp.float32,
                                      precision=lax.Precision.HIGHEST)
            s2 = s2 + lax.dot_general(OT, t * t, dn,
                                      preferred_element_type=jnp.float32,
                                      precision=lax.Precision.HIGHEST)
            cnt = cnt + jnp.sum(OT, axis=0).reshape(G, 1)
            return s1, s2, cnt

        zero_g = jnp.zeros((G, H), jnp.float32)
        s1, s2, cnt = lax.fori_loop(
            0, NBK, blk1, (zero_g, zero_g, jnp.zeros((G, 1), jnp.float32)))
        counts = jnp.maximum(cnt, 1.0)
        mean = s1 / counts
        mg = mean * gms_[...]
        var = s2 / counts - mg * (2.0 * mean - mg)
        inv = lax.rsqrt(var + 1e-5)
        bcast = jnp.concatenate([mg, inv], axis=1)

        def blk2(i, carry):
            sl = pl.ds(i * BN, BN)
            OT = onehot_blk(i)
            bc = jnp.dot(OT, bcast, preferred_element_type=jnp.float32,
                         precision=lax.Precision.HIGHEST)
            z2 = gw_[...] * (t_ref[sl, :] - bc[:, :H]) * bc[:, H:] + gb_[...]
            hn = jnp.maximum(z2, 0.0)
            if final:
                carry = carry + lax.dot_general(
                    OT, hn, (((0,), (0,)), ((), ())),
                    preferred_element_type=jnp.float32,
                    precision=lax.Precision.HIGHEST)
            else:
                out[sl, :] = hn
            return carry

        acc = lax.fori_loop(0, NBK, blk2, zero_g)
        if final:
            pooled = acc / counts
            out[...] = jnp.dot(pooled, wl[...],
                               preferred_element_type=jnp.float32) + bl_[...]

    args = [h, aggr, batch2.reshape(N, 1), W1, b1.reshape(1, H), W2, b2.reshape(1, H),
            gw.reshape(1, H), gb.reshape(1, H), gms.reshape(1, H)]
    if final:
        OUT = Wl.shape[1]
        args += [Wl, bl.reshape(1, OUT)]
        out_shape = jax.ShapeDtypeStruct((G, OUT), jnp.float32)
    else:
        out_shape = jax.ShapeDtypeStruct((N, H), jnp.float32)
    return pl.pallas_call(
        body,
        out_shape=out_shape,
        scratch_shapes=[pltpu.VMEM((N, H), jnp.float32)],
    )(*args)


def kernel(x, edge_index, edge_attr, batch,
           W1_0, b1_0, W2_0, b2_0, We_0, be_0, gn_w_0, gn_b_0, gn_ms_0,
           W1_1, b1_1, W2_1, b2_1, We_1, be_1, gn_w_1, gn_b_1, gn_ms_1,
           W1_2, b1_2, W2_2, b2_2, We_2, be_2, gn_w_2, gn_b_2, gn_ms_2,
           Wl, bl):
    N, H = x.shape
    E = edge_index.shape[1]
    G = 64

    src = edge_index[0]
    dst = edge_index[1]
    batch2 = batch.reshape(1, N).astype(jnp.int32)

    sc_aggr = _make_sc_aggr(N, E, H)

    h = x
    layers = [
        (We_0, be_0, W1_0, b1_0, W2_0, b2_0, gn_w_0, gn_b_0, gn_ms_0),
        (We_1, be_1, W1_1, b1_1, W2_1, b2_1, gn_w_1, gn_b_1, gn_ms_1),
    ]
    for (We, be, W1, b1, W2, b2, gw, gb, gms) in layers:
        e = _edge_mlp(edge_attr, We, be)
        aggr = sc_aggr(h, e, src, dst)
        h = _node_fused(h, aggr, batch2, G, W1, b1, W2, b2, gw, gb, gms)

    e = _edge_mlp(edge_attr, We_2, be_2)
    aggr = sc_aggr(h, e, src, dst)
    return _node_fused(h, aggr, batch2, G, W1_2, b1_2, W2_2, b2_2,
                       gn_w_2, gn_b_2, gn_ms_2, Wl, bl)


# speedup vs baseline: 1.0074x; 1.0074x over previous
import functools

import jax
import jax.numpy as jnp
from jax import lax
from jax.experimental import pallas as pl
from jax.experimental.pallas import tpu as pltpu
from jax.experimental.pallas import tpu_sc as plsc


@functools.lru_cache(maxsize=None)
def _make_sc_aggr(N, E, H):
    info = plsc.get_sparse_core_info()
    NC, NS, L = info.num_cores, info.num_subcores, info.num_lanes
    NW = NC * NS
    PER_W = E // NW
    C = 40
    NCH = PER_W // C
    assert PER_W % C == 0 and E % NW == 0
    NB = 3
    MB = 2
    D = 6
    UN = 6
    J = (NCH - 4) // UN
    J = NCH // UN
    TAIL = NCH - J * UN
    BLK = 40
    NBLK = N // BLK
    KMAX = (NBLK + NS - 1) // NS
    assert N % BLK == 0
    QS = H // L

    mesh = plsc.VectorSubcoreMesh(core_axis_name="c", subcore_axis_name="s")

    @functools.partial(
        pl.kernel,
        out_type=jax.ShapeDtypeStruct((NC, N, H), jnp.float32),
        mesh=mesh,
        scratch_types=[
            pltpu.VMEM((NB, C), jnp.int32),
            pltpu.VMEM((D, C), jnp.int32),
            pltpu.VMEM((NB, C, H), jnp.float32),
            pltpu.VMEM((NB, C, H), jnp.float32),
            pltpu.VMEM((MB, C, H), jnp.float32),
            pltpu.VMEM_SHARED((N, H), jnp.float32),
        ] + [pltpu.SemaphoreType.DMA] * 17,
    )
    def sc_aggr(h_hbm, e_hbm, src_hbm, dst_hbm, out_hbm,
                sring, dring, hring, ering, mring, accum, *sems):
        semsi = sems[0:3]
        semse = sems[3:6]
        semdi = sems[6:12]
        semg = sems[12:15]
        semsc = sems[15:17]
        cid = lax.axis_index("c")
        sid = lax.axis_index("s")
        wid = sid * NC + cid
        base = wid * PER_W

        zv = jnp.zeros((L,), jnp.float32)
        stage = mring.at[1]

        def zero_stage(r, carry):
            for q in range(QS):
                stage[r, pl.ds(q * L, L)] = zv
            return carry

        lax.fori_loop(0, BLK, zero_stage, 0)

        def zero_accum(k, carry):
            blk = sid + k * NS

            @pl.when(blk < NBLK)
            def _():
                pltpu.sync_copy(stage, accum.at[pl.ds(blk * BLK, BLK)])
            return carry

        def sidx_desc(c):
            b = c % NB if isinstance(c, int) else None
            return pltpu.make_async_copy(
                src_hbm.at[pl.ds(base + c * C, C)], sring.at[c % NB],
                semsi[c % NB])

        def didx_desc(c):
            return pltpu.make_async_copy(
                dst_hbm.at[pl.ds(base + c * C, C)], dring.at[c % D],
                semdi[c % D])

        def eload_desc(c):
            return pltpu.make_async_copy(
                e_hbm.at[pl.ds(base + c * C, C)], ering.at[c % NB],
                semse[c % NB])

        def gather_desc(c):
            return pltpu.make_async_copy(
                h_hbm.at[sring.at[c % NB]], hring.at[c % NB], semg[c % NB])

        def scat_desc(c):
            return pltpu.make_async_copy(
                mring.at[c % MB], accum.at[dring.at[c % D]], semsc[c % MB])

        def issue_loads(c):
            sidx_desc(c).start()
            didx_desc(c).start()
            eload_desc(c).start()

        def phase(c, first=False):
            if c2ok(c):
                sidx_desc(c + 2).wait()
                gather_desc(c + 2).start()
            gather_desc(c).wait()
            eload_desc(c).wait()
            if first:
                @pl.when(cval(c) >= MB)
                def _():
                    scat_desc(c - MB).wait()
            else:
                scat_desc(c - MB).wait()
            hb = hring.at[c % NB]
            eb = ering.at[c % NB]
            mb = mring.at[c % MB]
            def rowfn(r, rc):
                for q in range(QS):
                    sl = pl.ds(q * L, L)
                    mb[r, sl] = jnp.maximum(hb[r, sl] + eb[r, sl], 0.0)
                return rc

            lax.fori_loop(0, C, rowfn, 0)
            didx_desc(c).wait()
            scat_desc(c).start(add=True)
            if c3ok(c):
                issue_loads(c + 3)

        for c in range(NB):
            issue_loads(c)
        lax.fori_loop(0, KMAX, zero_accum, 0)
        plsc.subcore_barrier()
        for c in range(2):
            sidx_desc(c).wait()
            gather_desc(c).start()

        def loop_body(j, carry):
            c0 = j * UN
            for k in range(UN):
                phase(_Shifted(c0, k), first=(k < MB))
            return carry

        lax.fori_loop(0, J, loop_body, 0)
        for k in range(TAIL):
            phase(J * UN + k)

        for k in range(MB):
            scat_desc(NCH - MB + k).wait()
        plsc.subcore_barrier()

        def drain(k, carry):
            blk = sid + k * NS

            @pl.when(blk < NBLK)
            def _():
                r0 = blk * BLK
                pltpu.sync_copy(accum.at[pl.ds(r0, BLK)], stage)
                pltpu.sync_copy(stage, out_hbm.at[cid, pl.ds(r0, BLK)])
            return carry

        lax.fori_loop(0, KMAX, drain, 0)

    class _Shifted:
        def __init__(self, c0, k):
            self.c0 = c0
            self.k = k

        def __mod__(self, m):
            assert UN % m == 0
            return self.k % m

        def __add__(self, o):
            return _Shifted(self.c0, self.k + o)

        def __sub__(self, o):
            return _Shifted(self.c0, self.k - o)

        def __mul__(self, o):
            return (self.c0 + self.k) * o

        def __rmul__(self, o):
            return self.__mul__(o)

    def cval(c):
        return c.c0 + c.k if isinstance(c, _Shifted) else c

    def c2ok(c):
        if isinstance(c, _Shifted):
            return True
        return c + 2 < NCH

    def c3ok(c):
        if isinstance(c, _Shifted):
            return True
        return c + 3 < NCH

    return sc_aggr


def _edge_mlp3(edge_attr, We0, be0, We1, be1, We2, be2):
    E, DE = edge_attr.shape
    H = We0.shape[1]
    BE = 2000
    grid = (E // BE,)

    def body(ea, w0, b0, w1, b1, w2, b2, o0, o1, o2):
        a = ea[...]
        o0[...] = jnp.dot(a, w0[...], preferred_element_type=jnp.float32) + b0[...]
        o1[...] = jnp.dot(a, w1[...], preferred_element_type=jnp.float32) + b1[...]
        o2[...] = jnp.dot(a, w2[...], preferred_element_type=jnp.float32) + b2[...]

    full_w = pl.BlockSpec((DE, H), lambda i: (0, 0))
    full_b = pl.BlockSpec((1, H), lambda i: (0, 0))
    out_spec = pl.BlockSpec((BE, H), lambda i: (i, 0))
    return pl.pallas_call(
        body,
        grid=grid,
        in_specs=[pl.BlockSpec((BE, DE), lambda i: (i, 0)),
                  full_w, full_b, full_w, full_b, full_w, full_b],
        out_specs=[out_spec, out_spec, out_spec],
        out_shape=[jax.ShapeDtypeStruct((E, H), jnp.float32)] * 3,
    )(edge_attr, We0, be0.reshape(1, H), We1, be1.reshape(1, H),
      We2, be2.reshape(1, H))


def _node_fused(h, aggr, batch2, G, W1, b1, W2, b2, gw, gb, gms, Wl=None,
                bl=None):
    N, H = h.shape
    BN = 2000
    NBK = N // BN
    final = Wl is not None

    def body(*refs):
        if final:
            (h_ref, a_ref, bt_ref, w1, b1_, w2, b2_, gw_, gb_, gms_,
             wl, bl_, out, t_ref) = refs
        else:
            (h_ref, a_ref, bt_ref, w1, b1_, w2, b2_, gw_, gb_, gms_,
             out, t_ref) = refs

        def onehot_blk(i):
            bt = bt_ref[pl.ds(i * BN, BN), :]
            ot = lax.broadcasted_iota(jnp.int32, (BN, G), 1)
            return (ot == bt).astype(jnp.float32)

        def blk1(i, carry):
            s1, s2, cnt = carry
            sl = pl.ds(i * BN, BN)
            z = h_ref[sl, :] + a_ref[0, sl, :] + a_ref[1, sl, :]
            t = jnp.dot(
                jnp.maximum(jnp.dot(z, w1[...],
                                    preferred_element_type=jnp.float32)
                            + b1_[...], 0.0),
                w2[...], preferred_element_type=jnp.float32) + b2_[...]
            t_ref[sl, :] = t
            OT = onehot_blk(i)
            dn = (((0,), (0,)), ((), ()))
            s1 = s1 + lax.dot_general(OT, t, dn,
                                      preferred_element_type=jnp.float32,
                                      precision=lax.Precision.HIGHEST)
            s2 = s2 + lax.dot_general(OT, t * t, dn,
                                      preferred_element_type=jnp.float32,
                                      precision=lax.Precision.HIGHEST)
            cnt = cnt + jnp.sum(OT, axis=0).reshape(G, 1)
            return s1, s2, cnt

        zero_g = jnp.zeros((G, H), jnp.float32)
        s1, s2, cnt = lax.fori_loop(
            0, NBK, blk1, (zero_g, zero_g, jnp.zeros((G, 1), jnp.float32)))
        counts = jnp.maximum(cnt, 1.0)
        mean = s1 / counts
        mg = mean * gms_[...]
        var = s2 / counts - mg * (2.0 * mean - mg)
        inv = lax.rsqrt(var + 1e-5)
        bcast = jnp.concatenate([mg, inv], axis=1)

        def blk2(i, carry):
            sl = pl.ds(i * BN, BN)
            OT = onehot_blk(i)
            bc = jnp.dot(OT, bcast, preferred_element_type=jnp.float32,
                         precision=lax.Precision.HIGHEST)
            z2 = gw_[...] * (t_ref[sl, :] - bc[:, :H]) * bc[:, H:] + gb_[...]
            hn = jnp.maximum(z2, 0.0)
            if final:
                carry = carry + lax.dot_general(
                    OT, hn, (((0,), (0,)), ((), ())),
                    preferred_element_type=jnp.float32,
                    precision=lax.Precision.HIGHEST)
            else:
                out[sl, :] = hn
            return carry

        acc = lax.fori_loop(0, NBK, blk2, zero_g)
        if final:
            pooled = acc / counts
            out[...] = jnp.dot(pooled, wl[...],
                               preferred_element_type=jnp.float32) + bl_[...]

    args = [h, aggr, batch2.reshape(N, 1), W1, b1.reshape(1, H), W2, b2.reshape(1, H),
            gw.reshape(1, H), gb.reshape(1, H), gms.reshape(1, H)]
    if final:
        OUT = Wl.shape[1]
        args += [Wl, bl.reshape(1, OUT)]
        out_shape = jax.ShapeDtypeStruct((G, OUT), jnp.float32)
    else:
        out_shape = jax.ShapeDtypeStruct((N, H), jnp.float32)
    return pl.pallas_call(
        body,
        out_shape=out_shape,
        scratch_shapes=[pltpu.VMEM((N, H), jnp.float32)],
    )(*args)


def kernel(x, edge_index, edge_attr, batch,
           W1_0, b1_0, W2_0, b2_0, We_0, be_0, gn_w_0, gn_b_0, gn_ms_0,
           W1_1, b1_1, W2_1, b2_1, We_1, be_1, gn_w_1, gn_b_1, gn_ms_1,
           W1_2, b1_2, W2_2, b2_2, We_2, be_2, gn_w_2, gn_b_2, gn_ms_2,
           Wl, bl):
    N, H = x.shape
    E = edge_index.shape[1]
    G = 64

    src = edge_index[0]
    dst = edge_index[1]
    batch2 = batch.reshape(1, N).astype(jnp.int32)

    sc_aggr = _make_sc_aggr(N, E, H)

    h = x
    e0, e1, e2 = _edge_mlp3(edge_attr, We_0, be_0, We_1, be_1, We_2, be_2)
    layers = [
        (e0, W1_0, b1_0, W2_0, b2_0, gn_w_0, gn_b_0, gn_ms_0),
        (e1, W1_1, b1_1, W2_1, b2_1, gn_w_1, gn_b_1, gn_ms_1),
    ]
    for (e, W1, b1, W2, b2, gw, gb, gms) in layers:
        aggr = sc_aggr(h, e, src, dst)
        h = _node_fused(h, aggr, batch2, G, W1, b1, W2, b2, gw, gb, gms)

    aggr = sc_aggr(h, e2, src, dst)
    return _node_fused(h, aggr, batch2, G, W1_2, b1_2, W2_2, b2_2,
                       gn_w_2, gn_b_2, gn_ms_2, Wl, bl)
